# baseline (speedup 1.0000x reference)
import jax
import jax.numpy as jnp
from jax.experimental import pallas as pl
from jax.experimental.pallas import tpu as pltpu

B, S, RAW = 16384, 50, 109
EMB = 20
H = 512
NB = 10
OUT = 20
PACK = 4
PW = PACK * EMB

BP = 128
CH = 16
BM = 2048
NCHUNK = 2


def _pool_kernel(in_ref, mask_ref, rot_ref, q_ref, emb_ref, out_ref,
                 raw_tile, q_smem, sem):
    cp = pltpu.make_async_copy(q_ref, q_smem, sem)
    cp.start()
    cp.wait()

    def row_body(r, carry):
        for s in range(S):
            raw_tile[r, s] = emb_ref[q_smem[r, s], 0]
        return carry

    jax.lax.fori_loop(0, BP, row_body, 0)

    def clip(t):
        return jnp.clip(t, 1e-9, 1e9)

    iota_e = jax.lax.broadcasted_iota(jnp.int32, (1, 1, EMB), 2)
    iota_s = jax.lax.broadcasted_iota(jnp.int32, (1, S, 1), 1)
    for c in range(BP // CH):
        lo, hi = c * CH, (c + 1) * CH
        idx_c = rot_ref[lo:hi] + iota_e
        e_c = jnp.take_along_axis(raw_tile[lo:hi], idx_c, axis=2)
        x_c = jnp.concatenate([e_c, in_ref[lo:hi, :, 1:]], axis=2)
        seq_c = jnp.sum(mask_ref[lo:hi], axis=1, keepdims=True)
        m3 = iota_s < seq_c.astype(jnp.int32).reshape(CH, 1, 1)
        big = jnp.where(m3, 0.0, -1e30)
        msk = jnp.where(m3, 1.0, 0.0)
        x_sum = jnp.sum(x_c * msk, axis=1)
        x_mx = jnp.max(x_c + big, axis=1)
        x_mn = jnp.min(x_c - big, axis=1)
        out_ref[lo:hi, :] = jnp.concatenate([
            x_sum * (1.0 / seq_c), clip(x_mx), clip(x_mn), x_c[:, S - 1, :],
        ], axis=1)


def _mlp_kernel(h0_ref, w1_ref, b1_ref, w2_ref, b2_ref, wf_ref, bf_ref,
                out_ref, h_s):
    i = pl.program_id(1)

    @pl.when(i == 0)
    def _():
        h_s[...] = h0_ref[...]

    h = h_s[...]
    w1 = w1_ref[0]
    w2 = w2_ref[0]
    acc = None
    for c in range(NCHUNK):
        lo, hi = c * (4 * H // NCHUNK), (c + 1) * (4 * H // NCHUNK)
        tc = jnp.maximum(
            jnp.dot(h, w1[:, lo:hi], preferred_element_type=jnp.float32)
            + b1_ref[0, :, lo:hi], 0.0)
        part = jnp.dot(tc, w2[lo:hi, :], preferred_element_type=jnp.float32)
        acc = part if acc is None else acc + part
    h = h + jnp.maximum(acc + b2_ref[0], 0.0)
    h_s[...] = h

    @pl.when(i == NB - 1)
    def _():
        out_ref[...] = (
            jnp.dot(h, wf_ref[...], preferred_element_type=jnp.float32)
            + bf_ref[...])


def kernel(input, mask, embed, W1, b1, W2, b2, Wf, bf):
    ids = input[:, :, 0].astype(jnp.int32)
    q = ids >> 2
    rot3 = ((ids & 3) * EMB).reshape(B, S, 1)
    maskf = mask.astype(jnp.float32)
    vocab = embed.shape[0]
    emb3 = embed.reshape(vocab // PACK, 1, PW)

    h0 = pl.pallas_call(
        _pool_kernel,
        grid=(B // BP,),
        in_specs=[
            pl.BlockSpec((BP, S, RAW), lambda j: (j, 0, 0)),
            pl.BlockSpec((BP, S), lambda j: (j, 0)),
            pl.BlockSpec((BP, S, 1), lambda j: (j, 0, 0)),
            pl.BlockSpec((BP, S), lambda j: (j, 0)),
            pl.BlockSpec((vocab // PACK, 1, PW), lambda j: (0, 0, 0)),
        ],
        out_specs=pl.BlockSpec((BP, H), lambda j: (j, 0)),
        out_shape=jax.ShapeDtypeStruct((B, H), jnp.float32),
        scratch_shapes=[
            pltpu.VMEM((BP, S, PW), jnp.float32),
            pltpu.SMEM((BP, S), jnp.int32),
            pltpu.SemaphoreType.DMA,
        ],
        compiler_params=pltpu.CompilerParams(
            dimension_semantics=("parallel",),
            vmem_limit_bytes=56 * 2**20,
        ),
        name="pool",
    )(input, maskf, rot3, q, emb3)

    out = pl.pallas_call(
        _mlp_kernel,
        grid=(B // BM, NB),
        in_specs=[
            pl.BlockSpec((BM, H), lambda j, i: (j, 0)),
            pl.BlockSpec((1, H, 4 * H), lambda j, i: (i, 0, 0)),
            pl.BlockSpec((1, 1, 4 * H), lambda j, i: (i, 0, 0)),
            pl.BlockSpec((1, 4 * H, H), lambda j, i: (i, 0, 0)),
            pl.BlockSpec((1, 1, H), lambda j, i: (i, 0, 0)),
            pl.BlockSpec((H, OUT), lambda j, i: (0, 0)),
            pl.BlockSpec((1, OUT), lambda j, i: (0, 0)),
        ],
        out_specs=pl.BlockSpec((BM, OUT), lambda j, i: (j, 0)),
        out_shape=jax.ShapeDtypeStruct((B, OUT), jnp.float32),
        scratch_shapes=[pltpu.VMEM((BM, H), jnp.float32)],
        compiler_params=pltpu.CompilerParams(
            dimension_semantics=("parallel", "arbitrary"),
            vmem_limit_bytes=56 * 2**20,
        ),
        name="resmlp",
    )(h0, W1, b1.reshape(NB, 1, 4 * H), W2, b2.reshape(NB, 1, H),
      Wf, bf.reshape(1, OUT))
    return out


# speedup vs baseline: 1.0611x; 1.0611x over previous
import jax
import jax.numpy as jnp
from jax.experimental import pallas as pl
from jax.experimental.pallas import tpu as pltpu

B, S, RAW = 16384, 50, 109
EMB = 20
H = 512
NB = 10
OUT = 20
PACK = 4
PW = PACK * EMB

BP = 256
CH = 16
BM = 2048
NCHUNK = 2


def _pool_kernel(in_ref, mask_ref, rot_ref, q_ref, emb_ref, out_ref,
                 raw_tile, q_smem, sem):
    cp = pltpu.make_async_copy(q_ref, q_smem, sem)
    cp.start()
    cp.wait()

    def row_body(r, carry):
        for s in range(S):
            raw_tile[r, s] = emb_ref[q_smem[r, s], 0]
        return carry

    jax.lax.fori_loop(0, BP, row_body, 0)

    def clip(t):
        return jnp.clip(t, 1e-9, 1e9)

    iota_s = jax.lax.broadcasted_iota(jnp.int32, (1, S, 1), 1)
    for c in range(BP // CH):
        lo, hi = c * CH, (c + 1) * CH
        e_c = jnp.take_along_axis(raw_tile[lo:hi], rot_ref[lo:hi], axis=2)
        x_c = jnp.concatenate([e_c, in_ref[lo:hi, :, 1:]], axis=2)
        seq_c = jnp.sum(mask_ref[lo:hi], axis=1, keepdims=True)
        m3 = iota_s < seq_c.astype(jnp.int32).reshape(CH, 1, 1)
        big = jnp.where(m3, 0.0, -1e30)
        msk = jnp.where(m3, 1.0, 0.0)
        x_sum = jnp.sum(x_c * msk, axis=1)
        x_mx = jnp.max(x_c + big, axis=1)
        x_mn = jnp.min(x_c - big, axis=1)
        out_ref[lo:hi, :] = jnp.concatenate([
            x_sum * (1.0 / seq_c), clip(x_mx), clip(x_mn), x_c[:, S - 1, :],
        ], axis=1)


def _mlp_kernel(h0_ref, w1_ref, b1_ref, w2_ref, b2_ref, wf_ref, bf_ref,
                out_ref, h_s):
    i = pl.program_id(1)

    @pl.when(i == 0)
    def _():
        h_s[...] = h0_ref[...]

    h = h_s[...]
    w1 = w1_ref[0]
    w2 = w2_ref[0]
    acc = None
    for c in range(NCHUNK):
        lo, hi = c * (4 * H // NCHUNK), (c + 1) * (4 * H // NCHUNK)
        tc = jnp.maximum(
            jnp.dot(h, w1[:, lo:hi], preferred_element_type=jnp.float32)
            + b1_ref[0, :, lo:hi], 0.0)
        part = jnp.dot(tc, w2[lo:hi, :], preferred_element_type=jnp.float32)
        acc = part if acc is None else acc + part
    h = h + jnp.maximum(acc + b2_ref[0], 0.0)
    h_s[...] = h

    @pl.when(i == NB - 1)
    def _():
        out_ref[...] = (
            jnp.dot(h, wf_ref[...], preferred_element_type=jnp.float32)
            + bf_ref[...])


def kernel(input, mask, embed, W1, b1, W2, b2, Wf, bf):
    ids = input[:, :, 0].astype(jnp.int32)
    q = ids >> 2
    idx3 = ((ids & 3) * EMB)[:, :, None] + jnp.arange(EMB, dtype=jnp.int32)
    maskf = mask.astype(jnp.float32)
    vocab = embed.shape[0]
    emb3 = embed.reshape(vocab // PACK, 1, PW)

    h0 = pl.pallas_call(
        _pool_kernel,
        grid=(B // BP,),
        in_specs=[
            pl.BlockSpec((BP, S, RAW), lambda j: (j, 0, 0)),
            pl.BlockSpec((BP, S), lambda j: (j, 0)),
            pl.BlockSpec((BP, S, EMB), lambda j: (j, 0, 0)),
            pl.BlockSpec((BP, S), lambda j: (j, 0)),
            pl.BlockSpec((vocab // PACK, 1, PW), lambda j: (0, 0, 0)),
        ],
        out_specs=pl.BlockSpec((BP, H), lambda j: (j, 0)),
        out_shape=jax.ShapeDtypeStruct((B, H), jnp.float32),
        scratch_shapes=[
            pltpu.VMEM((BP, S, PW), jnp.float32),
            pltpu.SMEM((BP, S), jnp.int32),
            pltpu.SemaphoreType.DMA,
        ],
        compiler_params=pltpu.CompilerParams(
            dimension_semantics=("parallel",),
            vmem_limit_bytes=56 * 2**20,
        ),
        name="pool",
    )(input, maskf, idx3, q, emb3)

    out = pl.pallas_call(
        _mlp_kernel,
        grid=(B // BM, NB),
        in_specs=[
            pl.BlockSpec((BM, H), lambda j, i: (j, 0)),
            pl.BlockSpec((1, H, 4 * H), lambda j, i: (i, 0, 0)),
            pl.BlockSpec((1, 1, 4 * H), lambda j, i: (i, 0, 0)),
            pl.BlockSpec((1, 4 * H, H), lambda j, i: (i, 0, 0)),
            pl.BlockSpec((1, 1, H), lambda j, i: (i, 0, 0)),
            pl.BlockSpec((H, OUT), lambda j, i: (0, 0)),
            pl.BlockSpec((1, OUT), lambda j, i: (0, 0)),
        ],
        out_specs=pl.BlockSpec((BM, OUT), lambda j, i: (j, 0)),
        out_shape=jax.ShapeDtypeStruct((B, OUT), jnp.float32),
        scratch_shapes=[pltpu.VMEM((BM, H), jnp.float32)],
        compiler_params=pltpu.CompilerParams(
            dimension_semantics=("parallel", "arbitrary"),
            vmem_limit_bytes=56 * 2**20,
        ),
        name="resmlp",
    )(h0, W1, b1.reshape(NB, 1, 4 * H), W2, b2.reshape(NB, 1, H),
      Wf, bf.reshape(1, OUT))
    return out


# speedup vs baseline: 1.1281x; 1.0631x over previous
import jax
import jax.numpy as jnp
from jax.experimental import pallas as pl
from jax.experimental.pallas import tpu as pltpu

B, S, RAW = 16384, 50, 109
EMB = 20
H = 512
NB = 10
OUT = 20
PACK = 4
PW = PACK * EMB

BP = 256
CH = 16
BM = 2048
NCHUNK = 2


def _pool_kernel(in_ref, mask_ref, rot_ref, q_ref, qn_ref, emb_ref, out_ref,
                 raw_tile, q_smem, sem):
    j = pl.program_id(0)
    slot = jax.lax.rem(j, 2)

    @pl.when(j == 0)
    def _():
        pltpu.make_async_copy(q_ref, q_smem.at[0], sem.at[0]).start()

    @pl.when(j + 1 < B // BP)
    def _():
        nslot = 1 - slot
        pltpu.make_async_copy(qn_ref, q_smem.at[nslot], sem.at[nslot]).start()

    pltpu.make_async_copy(q_ref, q_smem.at[slot], sem.at[slot]).wait()

    def gather_from(qs):
        def row_body(r, carry):
            for s in range(S):
                raw_tile[r, s] = emb_ref[qs[r, s], 0]
            return carry
        jax.lax.fori_loop(0, BP, row_body, 0)

    @pl.when(slot == 0)
    def _():
        gather_from(q_smem.at[0])

    @pl.when(slot == 1)
    def _():
        gather_from(q_smem.at[1])

    def clip(t):
        return jnp.clip(t, 1e-9, 1e9)

    iota_s = jax.lax.broadcasted_iota(jnp.int32, (1, S, 1), 1)
    for c in range(BP // CH):
        lo, hi = c * CH, (c + 1) * CH
        e_c = jnp.take_along_axis(raw_tile[lo:hi], rot_ref[lo:hi], axis=2)
        x_c = jnp.concatenate([e_c, in_ref[lo:hi, :, 1:]], axis=2)
        seq_c = jnp.sum(mask_ref[lo:hi], axis=1, keepdims=True)
        m3 = iota_s < seq_c.astype(jnp.int32).reshape(CH, 1, 1)
        big = jnp.where(m3, 0.0, -1e30)
        msk = jnp.where(m3, 1.0, 0.0)
        x_sum = jnp.sum(x_c * msk, axis=1)
        x_mx = jnp.max(x_c + big, axis=1)
        x_mn = jnp.min(x_c - big, axis=1)
        out_ref[lo:hi, :] = jnp.concatenate([
            x_sum * (1.0 / seq_c), clip(x_mx), clip(x_mn), x_c[:, S - 1, :],
        ], axis=1)


def _mlp_kernel(h0_ref, w1_ref, b1_ref, w2_ref, b2_ref, wf_ref, bf_ref,
                out_ref, h_s):
    i = pl.program_id(1)

    @pl.when(i == 0)
    def _():
        h_s[...] = h0_ref[...]

    h = h_s[...]
    w1 = w1_ref[0]
    w2 = w2_ref[0]
    acc = None
    for c in range(NCHUNK):
        lo, hi = c * (4 * H // NCHUNK), (c + 1) * (4 * H // NCHUNK)
        tc = jnp.maximum(
            jnp.dot(h, w1[:, lo:hi], preferred_element_type=jnp.float32)
            + b1_ref[0, :, lo:hi], 0.0)
        part = jnp.dot(tc, w2[lo:hi, :], preferred_element_type=jnp.float32)
        acc = part if acc is None else acc + part
    h = h + jnp.maximum(acc + b2_ref[0], 0.0)
    h_s[...] = h

    @pl.when(i == NB - 1)
    def _():
        out_ref[...] = (
            jnp.dot(h, wf_ref[...], preferred_element_type=jnp.float32)
            + bf_ref[...])


def kernel(input, mask, embed, W1, b1, W2, b2, Wf, bf):
    ids = input[:, :, 0].astype(jnp.int32)
    q = ids >> 2
    idx3 = ((ids & 3) * EMB)[:, :, None] + jnp.arange(EMB, dtype=jnp.int32)
    maskf = mask.astype(jnp.float32)
    vocab = embed.shape[0]
    emb3 = embed.reshape(vocab // PACK, 1, PW)

    h0 = pl.pallas_call(
        _pool_kernel,
        grid=(B // BP,),
        in_specs=[
            pl.BlockSpec((BP, S, RAW), lambda j: (j, 0, 0)),
            pl.BlockSpec((BP, S), lambda j: (j, 0)),
            pl.BlockSpec((BP, S, EMB), lambda j: (j, 0, 0)),
            pl.BlockSpec((BP, S), lambda j: (j, 0)),
            pl.BlockSpec((BP, S), lambda j: (jnp.minimum(j + 1, B // BP - 1), 0)),
            pl.BlockSpec((vocab // PACK, 1, PW), lambda j: (0, 0, 0)),
        ],
        out_specs=pl.BlockSpec((BP, H), lambda j: (j, 0)),
        out_shape=jax.ShapeDtypeStruct((B, H), jnp.float32),
        scratch_shapes=[
            pltpu.VMEM((BP, S, PW), jnp.float32),
            pltpu.SMEM((2, BP, S), jnp.int32),
            pltpu.SemaphoreType.DMA((2,)),
        ],
        compiler_params=pltpu.CompilerParams(
            dimension_semantics=("parallel",),
            vmem_limit_bytes=56 * 2**20,
        ),
        name="pool",
    )(input, maskf, idx3, q, q, emb3)

    out = pl.pallas_call(
        _mlp_kernel,
        grid=(B // BM, NB),
        in_specs=[
            pl.BlockSpec((BM, H), lambda j, i: (j, 0)),
            pl.BlockSpec((1, H, 4 * H), lambda j, i: (i, 0, 0)),
            pl.BlockSpec((1, 1, 4 * H), lambda j, i: (i, 0, 0)),
            pl.BlockSpec((1, 4 * H, H), lambda j, i: (i, 0, 0)),
            pl.BlockSpec((1, 1, H), lambda j, i: (i, 0, 0)),
            pl.BlockSpec((H, OUT), lambda j, i: (0, 0)),
            pl.BlockSpec((1, OUT), lambda j, i: (0, 0)),
        ],
        out_specs=pl.BlockSpec((BM, OUT), lambda j, i: (j, 0)),
        out_shape=jax.ShapeDtypeStruct((B, OUT), jnp.float32),
        scratch_shapes=[pltpu.VMEM((BM, H), jnp.float32)],
        compiler_params=pltpu.CompilerParams(
            dimension_semantics=("parallel", "arbitrary"),
            vmem_limit_bytes=56 * 2**20,
        ),
        name="resmlp",
    )(h0, W1, b1.reshape(NB, 1, 4 * H), W2, b2.reshape(NB, 1, H),
      Wf, bf.reshape(1, OUT))
    return out


# speedup vs baseline: 1.1404x; 1.0109x over previous
import jax
import jax.numpy as jnp
from jax.experimental import pallas as pl
from jax.experimental.pallas import tpu as pltpu

B, S, RAW = 16384, 50, 109
EMB = 20
H = 512
NB = 10
OUT = 20
PACK = 4
PW = PACK * EMB

BP = 256
CH = 32
BM = 2048
NCHUNK = 2


def _pool_kernel(in_ref, mask_ref, rot_ref, q_ref, qn_ref, emb_ref, out_ref,
                 raw_tile, q_smem, sem):
    j = pl.program_id(0)
    slot = jax.lax.rem(j, 2)

    @pl.when(j == 0)
    def _():
        pltpu.make_async_copy(q_ref, q_smem.at[0], sem.at[0]).start()

    @pl.when(j + 1 < B // BP)
    def _():
        nslot = 1 - slot
        pltpu.make_async_copy(qn_ref, q_smem.at[nslot], sem.at[nslot]).start()

    pltpu.make_async_copy(q_ref, q_smem.at[slot], sem.at[slot]).wait()

    def gather_from(qs):
        def row_body(r, carry):
            for s in range(S):
                raw_tile[r, s] = emb_ref[qs[r, s], 0]
            return carry
        jax.lax.fori_loop(0, BP, row_body, 0)

    @pl.when(slot == 0)
    def _():
        gather_from(q_smem.at[0])

    @pl.when(slot == 1)
    def _():
        gather_from(q_smem.at[1])

    def clip(t):
        return jnp.clip(t, 1e-9, 1e9)

    iota_s = jax.lax.broadcasted_iota(jnp.int32, (1, S, 1), 1)
    for c in range(BP // CH):
        lo, hi = c * CH, (c + 1) * CH
        e_c = jnp.take_along_axis(raw_tile[lo:hi], rot_ref[lo:hi], axis=2)
        x_c = jnp.concatenate([e_c, in_ref[lo:hi, :, 1:]], axis=2)
        seq_c = jnp.sum(mask_ref[lo:hi], axis=1, keepdims=True)
        m3 = iota_s < seq_c.astype(jnp.int32).reshape(CH, 1, 1)
        big = jnp.where(m3, 0.0, -1e30)
        msk = jnp.where(m3, 1.0, 0.0)
        x_sum = jnp.sum(x_c * msk, axis=1)
        x_mx = jnp.max(x_c + big, axis=1)
        x_mn = jnp.min(x_c - big, axis=1)
        out_ref[lo:hi, :] = jnp.concatenate([
            x_sum * (1.0 / seq_c), clip(x_mx), clip(x_mn), x_c[:, S - 1, :],
        ], axis=1)


def _mlp_kernel(h0_ref, w1_ref, b1_ref, w2_ref, b2_ref, wf_ref, bf_ref,
                out_ref, h_s):
    i = pl.program_id(1)

    @pl.when(i == 0)
    def _():
        h_s[...] = h0_ref[...]

    h = h_s[...]
    w1 = w1_ref[0]
    w2 = w2_ref[0]
    acc = None
    for c in range(NCHUNK):
        lo, hi = c * (4 * H // NCHUNK), (c + 1) * (4 * H // NCHUNK)
        tc = jnp.maximum(
            jnp.dot(h, w1[:, lo:hi], preferred_element_type=jnp.float32)
            + b1_ref[0, :, lo:hi], 0.0)
        part = jnp.dot(tc, w2[lo:hi, :], preferred_element_type=jnp.float32)
        acc = part if acc is None else acc + part
    h = h + jnp.maximum(acc + b2_ref[0], 0.0)
    h_s[...] = h

    @pl.when(i == NB - 1)
    def _():
        out_ref[...] = (
            jnp.dot(h, wf_ref[...], preferred_element_type=jnp.float32)
            + bf_ref[...])


def kernel(input, mask, embed, W1, b1, W2, b2, Wf, bf):
    ids = input[:, :, 0].astype(jnp.int32)
    q = ids >> 2
    idx3 = ((ids & 3) * EMB)[:, :, None] + jnp.arange(EMB, dtype=jnp.int32)
    maskf = mask.astype(jnp.float32)
    vocab = embed.shape[0]
    emb3 = embed.reshape(vocab // PACK, 1, PW)

    h0 = pl.pallas_call(
        _pool_kernel,
        grid=(B // BP,),
        in_specs=[
            pl.BlockSpec((BP, S, RAW), lambda j: (j, 0, 0)),
            pl.BlockSpec((BP, S), lambda j: (j, 0)),
            pl.BlockSpec((BP, S, EMB), lambda j: (j, 0, 0)),
            pl.BlockSpec((BP, S), lambda j: (j, 0)),
            pl.BlockSpec((BP, S), lambda j: (jnp.minimum(j + 1, B // BP - 1), 0)),
            pl.BlockSpec((vocab // PACK, 1, PW), lambda j: (0, 0, 0)),
        ],
        out_specs=pl.BlockSpec((BP, H), lambda j: (j, 0)),
        out_shape=jax.ShapeDtypeStruct((B, H), jnp.float32),
        scratch_shapes=[
            pltpu.VMEM((BP, S, PW), jnp.float32),
            pltpu.SMEM((2, BP, S), jnp.int32),
            pltpu.SemaphoreType.DMA((2,)),
        ],
        compiler_params=pltpu.CompilerParams(
            dimension_semantics=("parallel",),
            vmem_limit_bytes=56 * 2**20,
        ),
        name="pool",
    )(input, maskf, idx3, q, q, emb3)

    out = pl.pallas_call(
        _mlp_kernel,
        grid=(B // BM, NB),
        in_specs=[
            pl.BlockSpec((BM, H), lambda j, i: (j, 0)),
            pl.BlockSpec((1, H, 4 * H), lambda j, i: (i, 0, 0)),
            pl.BlockSpec((1, 1, 4 * H), lambda j, i: (i, 0, 0)),
            pl.BlockSpec((1, 4 * H, H), lambda j, i: (i, 0, 0)),
            pl.BlockSpec((1, 1, H), lambda j, i: (i, 0, 0)),
            pl.BlockSpec((H, OUT), lambda j, i: (0, 0)),
            pl.BlockSpec((1, OUT), lambda j, i: (0, 0)),
        ],
        out_specs=pl.BlockSpec((BM, OUT), lambda j, i: (j, 0)),
        out_shape=jax.ShapeDtypeStruct((B, OUT), jnp.float32),
        scratch_shapes=[pltpu.VMEM((BM, H), jnp.float32)],
        compiler_params=pltpu.CompilerParams(
            dimension_semantics=("parallel", "arbitrary"),
            vmem_limit_bytes=56 * 2**20,
        ),
        name="resmlp",
    )(h0, W1, b1.reshape(NB, 1, 4 * H), W2, b2.reshape(NB, 1, H),
      Wf, bf.reshape(1, OUT))
    return out
